# baseline (speedup 1.0000x reference)
import functools
import jax
import jax.numpy as jnp
from jax import lax
from jax.experimental import pallas as pl
from jax.experimental.pallas import tpu as pltpu
from jax.experimental.pallas import tpu_sc as plsc

NC = 2
NS = 16
LANES = 16


def kernel(x, pos_table):
    B, S, D = x.shape
    NW = NC * NS
    rows_per_w = S // NW
    R = 16
    n_chunks = rows_per_w // R
    nc2 = n_chunks // 2

    mesh = plsc.VectorSubcoreMesh(
        core_axis_name="c", subcore_axis_name="s", num_cores=NC, num_subcores=NS
    )

    @functools.partial(
        pl.kernel,
        mesh=mesh,
        out_type=jax.ShapeDtypeStruct((B, S, D), x.dtype),
        scratch_types=[
            pltpu.VMEM((2, R, D), jnp.float32),
            pltpu.VMEM((2, B, R, D), jnp.float32),
            pltpu.SemaphoreType.DMA((2,)),
            pltpu.SemaphoreType.DMA((2, B)),
            pltpu.SemaphoreType.DMA((2, B)),
        ],
    )
    def sc_add(x_hbm, pos_hbm, out_hbm, pbuf, xbuf, psem, insem, outsem):
        wid = lax.axis_index("s") * NC + lax.axis_index("c")
        base = wid * rows_per_w

        def pos_copy(c, par):
            return pltpu.make_async_copy(
                pos_hbm.at[pl.ds(base + c * R, R), :], pbuf.at[par], psem.at[par]
            )

        def x_copy(c, par, b):
            return pltpu.make_async_copy(
                x_hbm.at[b, pl.ds(base + c * R, R), :],
                xbuf.at[par, b],
                insem.at[par, b],
            )

        def out_copy(c, par, b):
            return pltpu.make_async_copy(
                xbuf.at[par, b],
                out_hbm.at[b, pl.ds(base + c * R, R), :],
                outsem.at[par, b],
            )

        pos_copy(0, 0).start()
        for b in range(B):
            x_copy(0, 0, b).start()

        def element(c, c2, par):
            pos_copy(c, par).wait()
            if par == 0:
                pos_copy(c + 1, 1).start()
            else:
                @pl.when(c2 < nc2 - 1)
                def _():
                    pos_copy(c + 1, 0).start()

            for b in range(B):
                x_copy(c, par, b).wait()

                def row_body(r, rcarry):
                    for j in range(D // LANES):
                        sl = pl.ds(j * LANES, LANES)
                        xbuf[par, b, r, sl] = xbuf[par, b, r, sl] + pbuf[par, r, sl]
                    return rcarry

                lax.fori_loop(0, R, row_body, 0)
                out_copy(c, par, b).start()

            for b in range(B):
                if par == 0:
                    @pl.when(c2 > 0)
                    def _():
                        out_copy(c - 1, 1, b).wait()
                    x_copy(c + 1, 1, b).start()
                else:
                    out_copy(c - 1, 0, b).wait()

                    @pl.when(c2 < nc2 - 1)
                    def _():
                        x_copy(c + 1, 0, b).start()

        def pair_body(c2, carry):
            element(2 * c2, c2, 0)
            element(2 * c2 + 1, c2, 1)
            return carry

        lax.fori_loop(0, nc2, pair_body, 0)

        for b in range(B):
            out_copy(n_chunks - 1, 1, b).wait()

    return sc_add(x, pos_table)
